# Initial kernel scaffold; baseline (speedup 1.0000x reference)
import jax
import jax.numpy as jnp
from jax.experimental import pallas as pl


def kernel(data, edge_index, W1, b1, g1, be1, W2, b2, g2, be2, W3, b3):
    raise NotImplementedError("write your pallas kernel here")


# speedup vs baseline: 22.8600x; 22.8600x over previous
import functools

import jax
import jax.numpy as jnp
from jax import lax
from jax.experimental import pallas as pl
from jax.experimental.pallas import tpu as pltpu
from jax.experimental.pallas import tpu_sc as plsc

N = 10000
E = 320000
H = 64
C = 40

NC = 2
NS = 16
NW = NC * NS
EPW = E // NW
WIN = 80
NWIN = EPW // WIN
NP = 10240
SL = NP // NS

_mesh = plsc.VectorSubcoreMesh(core_axis_name="c", subcore_axis_name="s")
_sc_params = pltpu.CompilerParams(use_tc_tiling_on_sc=False)


@functools.partial(
    pl.kernel,
    out_type=jax.ShapeDtypeStruct((NC, 1, NP), jnp.float32),
    mesh=_mesh,
    scratch_types=[
        pltpu.VMEM((NWIN, 1, WIN), jnp.int32),
        pltpu.VMEM((WIN,), jnp.float32),
        pltpu.VMEM_SHARED((NP,), jnp.float32),
    ],
    compiler_params=_sc_params,
)
def _sc_degree(colr_hbm, zeros_hbm, ones_hbm, degp_hbm, colv, onesv, deg_sh):
    c = lax.axis_index("c")
    s = lax.axis_index("s")
    w = c * NS + s
    pltpu.sync_copy(colr_hbm.at[w], colv)
    pltpu.sync_copy(ones_hbm, onesv)
    pltpu.sync_copy(zeros_hbm.at[pl.ds(s * SL, SL)],
                    deg_sh.at[pl.ds(s * SL, SL)])
    plsc.subcore_barrier()

    def body(j, carry):
        pltpu.sync_copy(onesv, deg_sh.at[colv.at[j, 0]], add=True)
        return carry

    lax.fori_loop(0, NWIN, body, 0)
    plsc.subcore_barrier()
    pltpu.sync_copy(deg_sh.at[pl.ds(s * SL, SL)],
                    degp_hbm.at[c, 0, pl.ds(s * SL, SL)])


@functools.partial(
    pl.kernel,
    out_type=jax.ShapeDtypeStruct((NC, NP, H), jnp.float32),
    mesh=_mesh,
    scratch_types=[
        pltpu.VMEM((NWIN, 1, WIN), jnp.int32),
        pltpu.VMEM((NWIN, 1, WIN), jnp.int32),
        pltpu.VMEM((WIN, H), jnp.float32),
        pltpu.VMEM_SHARED((NP, H), jnp.float32),
        pltpu.SemaphoreType.DMA,
    ],
    compiler_params=_sc_params,
)
def _sc_propagate(h_hbm, rowr_hbm, colr_hbm, zeros_hbm, outp_hbm,
                  rowv, colv, gbuf, acc_sh, sem):
    c = lax.axis_index("c")
    s = lax.axis_index("s")
    w = c * NS + s
    pltpu.sync_copy(rowr_hbm.at[w], rowv)
    pltpu.sync_copy(colr_hbm.at[w], colv)
    pltpu.sync_copy(zeros_hbm.at[pl.ds(s * SL, SL)],
                    acc_sh.at[pl.ds(s * SL, SL)])
    plsc.subcore_barrier()

    def body(j, carry):
        pltpu.async_copy(h_hbm.at[rowv.at[j, 0]], gbuf, sem).wait()
        pltpu.sync_copy(gbuf, acc_sh.at[colv.at[j, 0]], add=True)
        return carry

    lax.fori_loop(0, NWIN, body, 0)
    plsc.subcore_barrier()
    pltpu.sync_copy(acc_sh.at[pl.ds(s * SL, SL)],
                    outp_hbm.at[c, pl.ds(s * SL, SL)])


def _tc1_body(degp_ref, data_ref, w1_ref, hp_ref):
    dinv = lax.rsqrt(degp_ref[0] + degp_ref[1] + 1.0)
    h = jnp.dot(data_ref[...], w1_ref[...], preferred_element_type=jnp.float32)
    hp_ref[...] = h * dinv


def _tc2_body(degp_ref, s1p_ref, hp1_ref, b1_ref, g1_ref, be1_ref, w2_ref,
              hp2_ref):
    dinv = lax.rsqrt(degp_ref[0] + degp_ref[1] + 1.0)
    s1 = (s1p_ref[0] + s1p_ref[1] + hp1_ref[...]) * dinv + b1_ref[...]
    y = jnp.maximum(s1, 0.0)
    mean = jnp.mean(y, axis=0, keepdims=True)
    cent = y - mean
    var = jnp.mean(cent * cent, axis=0, keepdims=True)
    x1 = jnp.maximum(cent * lax.rsqrt(var + 1e-5) * g1_ref[...] + be1_ref[...],
                     0.0)
    hp2_ref[...] = jnp.dot(x1, w2_ref[...],
                           preferred_element_type=jnp.float32) * dinv


def _tc3_body(degp_ref, s2p_ref, hp2_ref, b2_ref, g2_ref, be2_ref, w3_ref,
              b3_ref, out_ref):
    dinv = lax.rsqrt(degp_ref[0] + degp_ref[1] + 1.0)
    s2 = (s2p_ref[0] + s2p_ref[1] + hp2_ref[...]) * dinv + b2_ref[...]
    y = jnp.maximum(s2, 0.0)
    mean = jnp.mean(y, axis=0, keepdims=True)
    cent = y - mean
    var = jnp.mean(cent * cent, axis=0, keepdims=True)
    x2 = cent * lax.rsqrt(var + 1e-5) * g2_ref[...] + be2_ref[...]
    out_ref[...] = jnp.maximum(
        jnp.dot(x2, w3_ref[...], preferred_element_type=jnp.float32)
        + b3_ref[...], 0.0)


def kernel(data, edge_index, W1, b1, g1, be1, W2, b2, g2, be2, W3, b3):
    rowr = edge_index[0].reshape(NW, NWIN, 1, WIN)
    colr = edge_index[1].reshape(NW, NWIN, 1, WIN)
    zeros_np = jnp.zeros((NP,), jnp.float32)
    ones_w = jnp.ones((WIN,), jnp.float32)
    zeros_nph = jnp.zeros((NP, H), jnp.float32)

    degp = _sc_degree(colr, zeros_np, ones_w)
    degp3 = degp[:, 0, :N, None]

    hp1 = pl.pallas_call(
        _tc1_body,
        out_shape=jax.ShapeDtypeStruct((N, H), jnp.float32),
    )(degp3, data, W1)

    s1p = _sc_propagate(hp1, rowr, colr, zeros_nph)[:, :N, :]

    hp2 = pl.pallas_call(
        _tc2_body,
        out_shape=jax.ShapeDtypeStruct((N, H), jnp.float32),
    )(degp3, s1p, hp1, b1.reshape(1, H), g1.reshape(1, H), be1.reshape(1, H),
      W2)

    s2p = _sc_propagate(hp2, rowr, colr, zeros_nph)[:, :N, :]

    out = pl.pallas_call(
        _tc3_body,
        out_shape=jax.ShapeDtypeStruct((N, C), jnp.float32),
    )(degp3, s2p, hp2, b2.reshape(1, H), g2.reshape(1, H), be2.reshape(1, H),
      W3, b3.reshape(1, C))
    return out


# speedup vs baseline: 29.3459x; 1.2837x over previous
import functools

import jax
import jax.numpy as jnp
from jax import lax
from jax.experimental import pallas as pl
from jax.experimental.pallas import tpu as pltpu
from jax.experimental.pallas import tpu_sc as plsc

N = 10000
E = 320000
H = 64
C = 40

NC = 2
NS = 16
NW = NC * NS
EPW = E // NW
WIN = 80
NWIN = EPW // WIN
NP = 10240
SL = NP // NS

_mesh = plsc.VectorSubcoreMesh(core_axis_name="c", subcore_axis_name="s")
_sc_params = pltpu.CompilerParams(use_tc_tiling_on_sc=False)


@functools.partial(
    pl.kernel,
    out_type=jax.ShapeDtypeStruct((NC, 1, NP), jnp.float32),
    mesh=_mesh,
    scratch_types=[
        pltpu.VMEM((NWIN, 1, WIN), jnp.int32),
        pltpu.VMEM((WIN,), jnp.float32),
        pltpu.VMEM_SHARED((NP,), jnp.float32),
    ],
    compiler_params=_sc_params,
)
def _sc_degree(colr_hbm, zeros_hbm, ones_hbm, degp_hbm, colv, onesv, deg_sh):
    c = lax.axis_index("c")
    s = lax.axis_index("s")
    w = c * NS + s
    pltpu.sync_copy(colr_hbm.at[w], colv)
    pltpu.sync_copy(ones_hbm, onesv)
    pltpu.sync_copy(zeros_hbm.at[pl.ds(s * SL, SL)],
                    deg_sh.at[pl.ds(s * SL, SL)])
    plsc.subcore_barrier()

    def body(j, carry):
        pltpu.sync_copy(onesv, deg_sh.at[colv.at[j, 0]], add=True)
        return carry

    lax.fori_loop(0, NWIN, body, 0)
    plsc.subcore_barrier()
    pltpu.sync_copy(deg_sh.at[pl.ds(s * SL, SL)],
                    degp_hbm.at[c, 0, pl.ds(s * SL, SL)])


@functools.partial(
    pl.kernel,
    out_type=jax.ShapeDtypeStruct((NC, NP, H), jnp.float32),
    mesh=_mesh,
    scratch_types=[
        pltpu.VMEM((NWIN, 1, WIN), jnp.int32),
        pltpu.VMEM((NWIN, 1, WIN), jnp.int32),
        pltpu.VMEM((WIN, H), jnp.float32),
        pltpu.VMEM_SHARED((NP, H), jnp.float32),
        pltpu.VMEM_SHARED((NP, H), jnp.float32),
        pltpu.SemaphoreType.DMA,
    ],
    compiler_params=_sc_params,
)
def _sc_propagate(h_hbm, rowr_hbm, colr_hbm, zeros_hbm, outp_hbm,
                  rowv, colv, gbuf, acc_sh, h_sh, sem):
    c = lax.axis_index("c")
    s = lax.axis_index("s")
    w = c * NS + s
    pltpu.sync_copy(rowr_hbm.at[w], rowv)
    pltpu.sync_copy(colr_hbm.at[w], colv)
    pltpu.sync_copy(zeros_hbm.at[pl.ds(s * SL, SL)],
                    acc_sh.at[pl.ds(s * SL, SL)])
    pltpu.sync_copy(h_hbm.at[pl.ds(s * SL, SL)],
                    h_sh.at[pl.ds(s * SL, SL)])
    plsc.subcore_barrier()

    def body(j, carry):
        pltpu.async_copy(h_sh.at[rowv.at[j, 0]], gbuf, sem).wait()
        pltpu.sync_copy(gbuf, acc_sh.at[colv.at[j, 0]], add=True)
        return carry

    lax.fori_loop(0, NWIN, body, 0)
    plsc.subcore_barrier()
    pltpu.sync_copy(acc_sh.at[pl.ds(s * SL, SL)],
                    outp_hbm.at[c, pl.ds(s * SL, SL)])


def _tc1_body(degp_ref, data_ref, w1_ref, hp_ref):
    dinv = lax.rsqrt(degp_ref[0] + degp_ref[1] + 1.0)
    h = jnp.dot(data_ref[...], w1_ref[...], preferred_element_type=jnp.float32)
    hp_ref[pl.ds(0, N)] = h * dinv


def _tc2_body(degp_ref, s1p_ref, hp1_ref, b1_ref, g1_ref, be1_ref, w2_ref,
              hp2_ref):
    dinv = lax.rsqrt(degp_ref[0] + degp_ref[1] + 1.0)
    hp1 = hp1_ref[pl.ds(0, N)]
    s1 = (s1p_ref[0, pl.ds(0, N)] + s1p_ref[1, pl.ds(0, N)] + hp1) * dinv \
        + b1_ref[...]
    y = jnp.maximum(s1, 0.0)
    mean = jnp.mean(y, axis=0, keepdims=True)
    cent = y - mean
    var = jnp.mean(cent * cent, axis=0, keepdims=True)
    x1 = jnp.maximum(cent * lax.rsqrt(var + 1e-5) * g1_ref[...] + be1_ref[...],
                     0.0)
    hp2_ref[pl.ds(0, N)] = jnp.dot(x1, w2_ref[...],
                                   preferred_element_type=jnp.float32) * dinv


def _tc3_body(degp_ref, s2p_ref, hp2_ref, b2_ref, g2_ref, be2_ref, w3_ref,
              b3_ref, out_ref):
    dinv = lax.rsqrt(degp_ref[0] + degp_ref[1] + 1.0)
    hp2 = hp2_ref[pl.ds(0, N)]
    s2 = (s2p_ref[0, pl.ds(0, N)] + s2p_ref[1, pl.ds(0, N)] + hp2) * dinv \
        + b2_ref[...]
    y = jnp.maximum(s2, 0.0)
    mean = jnp.mean(y, axis=0, keepdims=True)
    cent = y - mean
    var = jnp.mean(cent * cent, axis=0, keepdims=True)
    x2 = cent * lax.rsqrt(var + 1e-5) * g2_ref[...] + be2_ref[...]
    out_ref[...] = jnp.maximum(
        jnp.dot(x2, w3_ref[...], preferred_element_type=jnp.float32)
        + b3_ref[...], 0.0)


def kernel(data, edge_index, W1, b1, g1, be1, W2, b2, g2, be2, W3, b3):
    rowr = edge_index[0].reshape(NW, NWIN, 1, WIN)
    colr = edge_index[1].reshape(NW, NWIN, 1, WIN)
    zeros_np = jnp.zeros((NP,), jnp.float32)
    ones_w = jnp.ones((WIN,), jnp.float32)
    zeros_nph = jnp.zeros((NP, H), jnp.float32)

    degp = _sc_degree(colr, zeros_np, ones_w)
    degp3 = degp[:, 0, :N, None]

    hp1 = pl.pallas_call(
        _tc1_body,
        out_shape=jax.ShapeDtypeStruct((NP, H), jnp.float32),
    )(degp3, data, W1)

    s1p = _sc_propagate(hp1, rowr, colr, zeros_nph)

    hp2 = pl.pallas_call(
        _tc2_body,
        out_shape=jax.ShapeDtypeStruct((NP, H), jnp.float32),
    )(degp3, s1p, hp1, b1.reshape(1, H), g1.reshape(1, H), be1.reshape(1, H),
      W2)

    s2p = _sc_propagate(hp2, rowr, colr, zeros_nph)

    out = pl.pallas_call(
        _tc3_body,
        out_shape=jax.ShapeDtypeStruct((N, C), jnp.float32),
    )(degp3, s2p, hp2, b2.reshape(1, H), g2.reshape(1, H), be2.reshape(1, H),
      W3, b3.reshape(1, C))
    return out


# speedup vs baseline: 32.0551x; 1.0923x over previous
import functools

import jax
import jax.numpy as jnp
from jax import lax
from jax.experimental import pallas as pl
from jax.experimental.pallas import tpu as pltpu
from jax.experimental.pallas import tpu_sc as plsc

N = 10000
E = 320000
H = 64
C = 40

NC = 2
NS = 16
NW = NC * NS
EPW = E // NW
WIN = 80
NWIN = EPW // WIN
NP = 10240
SL = NP // NS
K = 5

_mesh = plsc.VectorSubcoreMesh(core_axis_name="c", subcore_axis_name="s")
_sc_params = pltpu.CompilerParams(use_tc_tiling_on_sc=False)


@functools.partial(
    pl.kernel,
    out_type=jax.ShapeDtypeStruct((NC, 1, NP), jnp.float32),
    mesh=_mesh,
    scratch_types=[
        pltpu.VMEM((NWIN, 1, WIN), jnp.int32),
        pltpu.VMEM((WIN,), jnp.float32),
        pltpu.VMEM_SHARED((NP,), jnp.float32),
    ],
    compiler_params=_sc_params,
)
def _sc_degree(colr_hbm, zeros_hbm, ones_hbm, degp_hbm, colv, onesv, deg_sh):
    c = lax.axis_index("c")
    s = lax.axis_index("s")
    w = c * NS + s
    pltpu.sync_copy(colr_hbm.at[w], colv)
    pltpu.sync_copy(ones_hbm, onesv)
    pltpu.sync_copy(zeros_hbm.at[pl.ds(s * SL, SL)],
                    deg_sh.at[pl.ds(s * SL, SL)])
    plsc.subcore_barrier()

    def body(j, carry):
        pltpu.sync_copy(onesv, deg_sh.at[colv.at[j, 0]], add=True)
        return carry

    lax.fori_loop(0, NWIN, body, 0)
    plsc.subcore_barrier()
    pltpu.sync_copy(deg_sh.at[pl.ds(s * SL, SL)],
                    degp_hbm.at[c, 0, pl.ds(s * SL, SL)])


@functools.partial(
    pl.kernel,
    out_type=jax.ShapeDtypeStruct((NC, NP, H), jnp.float32),
    mesh=_mesh,
    scratch_types=[
        pltpu.VMEM((NWIN, 1, WIN), jnp.int32),
        pltpu.VMEM((NWIN, 1, WIN), jnp.int32),
        pltpu.VMEM((K, WIN, H), jnp.float32),
        pltpu.VMEM_SHARED((NP, H), jnp.float32),
        pltpu.VMEM_SHARED((NP, H), jnp.float32),
        pltpu.SemaphoreType.DMA((K,)),
        pltpu.SemaphoreType.DMA((K,)),
    ],
    compiler_params=_sc_params,
)
def _sc_propagate(h_hbm, rowr_hbm, colr_hbm, zeros_hbm, outp_hbm,
                  rowv, colv, gbuf, acc_sh, h_sh, gsem, ssem):
    c = lax.axis_index("c")
    s = lax.axis_index("s")
    w = c * NS + s
    pltpu.sync_copy(rowr_hbm.at[w], rowv)
    pltpu.sync_copy(colr_hbm.at[w], colv)
    pltpu.sync_copy(zeros_hbm.at[pl.ds(s * SL, SL)],
                    acc_sh.at[pl.ds(s * SL, SL)])
    pltpu.sync_copy(h_hbm.at[pl.ds(s * SL, SL)],
                    h_sh.at[pl.ds(s * SL, SL)])
    plsc.subcore_barrier()

    for k in range(K):
        pltpu.async_copy(h_sh.at[rowv.at[k, 0]], gbuf.at[k], gsem.at[k])

    def body(i, carry):
        for k in range(K):
            j = K * i + k
            pltpu.make_async_copy(h_sh.at[rowv.at[j, 0]], gbuf.at[k],
                                  gsem.at[k]).wait()
            pltpu.async_copy(gbuf.at[k], acc_sh.at[colv.at[j, 0]],
                             ssem.at[k], add=True)
        for k in range(K):
            j = K * i + k
            jn = j + K
            pltpu.make_async_copy(gbuf.at[k], acc_sh.at[colv.at[j, 0]],
                                  ssem.at[k]).wait()

            @pl.when(jn < NWIN)
            def _():
                pltpu.async_copy(h_sh.at[rowv.at[jn, 0]], gbuf.at[k],
                                 gsem.at[k])
        return carry

    lax.fori_loop(0, NWIN // K, body, 0)
    plsc.subcore_barrier()
    pltpu.sync_copy(acc_sh.at[pl.ds(s * SL, SL)],
                    outp_hbm.at[c, pl.ds(s * SL, SL)])


def _tc1_body(degp_ref, data_ref, w1_ref, hp_ref):
    dinv = lax.rsqrt(degp_ref[0] + degp_ref[1] + 1.0)
    h = jnp.dot(data_ref[...], w1_ref[...], preferred_element_type=jnp.float32)
    hp_ref[pl.ds(0, N)] = h * dinv


def _tc2_body(degp_ref, s1p_ref, hp1_ref, b1_ref, g1_ref, be1_ref, w2_ref,
              hp2_ref):
    dinv = lax.rsqrt(degp_ref[0] + degp_ref[1] + 1.0)
    hp1 = hp1_ref[pl.ds(0, N)]
    s1 = (s1p_ref[0, pl.ds(0, N)] + s1p_ref[1, pl.ds(0, N)] + hp1) * dinv \
        + b1_ref[...]
    y = jnp.maximum(s1, 0.0)
    mean = jnp.mean(y, axis=0, keepdims=True)
    cent = y - mean
    var = jnp.mean(cent * cent, axis=0, keepdims=True)
    x1 = jnp.maximum(cent * lax.rsqrt(var + 1e-5) * g1_ref[...] + be1_ref[...],
                     0.0)
    hp2_ref[pl.ds(0, N)] = jnp.dot(x1, w2_ref[...],
                                   preferred_element_type=jnp.float32) * dinv


def _tc3_body(degp_ref, s2p_ref, hp2_ref, b2_ref, g2_ref, be2_ref, w3_ref,
              b3_ref, out_ref):
    dinv = lax.rsqrt(degp_ref[0] + degp_ref[1] + 1.0)
    hp2 = hp2_ref[pl.ds(0, N)]
    s2 = (s2p_ref[0, pl.ds(0, N)] + s2p_ref[1, pl.ds(0, N)] + hp2) * dinv \
        + b2_ref[...]
    y = jnp.maximum(s2, 0.0)
    mean = jnp.mean(y, axis=0, keepdims=True)
    cent = y - mean
    var = jnp.mean(cent * cent, axis=0, keepdims=True)
    x2 = cent * lax.rsqrt(var + 1e-5) * g2_ref[...] + be2_ref[...]
    out_ref[...] = jnp.maximum(
        jnp.dot(x2, w3_ref[...], preferred_element_type=jnp.float32)
        + b3_ref[...], 0.0)


def kernel(data, edge_index, W1, b1, g1, be1, W2, b2, g2, be2, W3, b3):
    rowr = edge_index[0].reshape(NW, NWIN, 1, WIN)
    colr = edge_index[1].reshape(NW, NWIN, 1, WIN)
    zeros_np = jnp.zeros((NP,), jnp.float32)
    ones_w = jnp.ones((WIN,), jnp.float32)
    zeros_nph = jnp.zeros((NP, H), jnp.float32)

    degp = _sc_degree(colr, zeros_np, ones_w)
    degp3 = degp[:, 0, :N, None]

    hp1 = pl.pallas_call(
        _tc1_body,
        out_shape=jax.ShapeDtypeStruct((NP, H), jnp.float32),
    )(degp3, data, W1)

    s1p = _sc_propagate(hp1, rowr, colr, zeros_nph)

    hp2 = pl.pallas_call(
        _tc2_body,
        out_shape=jax.ShapeDtypeStruct((NP, H), jnp.float32),
    )(degp3, s1p, hp1, b1.reshape(1, H), g1.reshape(1, H), be1.reshape(1, H),
      W2)

    s2p = _sc_propagate(hp2, rowr, colr, zeros_nph)

    out = pl.pallas_call(
        _tc3_body,
        out_shape=jax.ShapeDtypeStruct((N, C), jnp.float32),
    )(degp3, s2p, hp2, b2.reshape(1, H), g2.reshape(1, H), be2.reshape(1, H),
      W3, b3.reshape(1, C))
    return out
